# Initial kernel scaffold; baseline (speedup 1.0000x reference)
import jax
import jax.numpy as jnp
from jax.experimental import pallas as pl


def kernel(x, edge_index, batch, W0, b0, W1, b1, W2, b2):
    raise NotImplementedError("write your pallas kernel here")


# speedup vs baseline: 71.0731x; 71.0731x over previous
import functools

import jax
import jax.numpy as jnp
from jax import lax
from jax.experimental import pallas as pl
from jax.experimental.pallas import tpu as pltpu
from jax.experimental.pallas import tpu_sc as plsc

N_NODES = 10000
N_EDGES = 320000
N_GRAPHS = 64
D = 128

NC = 2
NS = 16
NW = NC * NS
EPW = N_EDGES // NW
CH = 2000
NCHUNK = EPW // CH
CVREG = CH // 16
ROWS_PER_HALF = (CVREG + 7) // 8
IDX_ROWS = 2 * ROWS_PER_HALF
C_SIZE = 2 * N_GRAPHS * N_NODES
S_SIZE = C_SIZE + 128
ZONE = C_SIZE // NS
ZCHUNK = 4096


def _sc_body(src_hbm, dst_hbm, batch_hbm, out_hbm,
             hist_sh, src_v, dst_v, batch_v, idx_v, ones_v, zb_v):
    c = lax.axis_index("c")
    s = lax.axis_index("s")
    wid = s * NC + c
    base = wid * EPW

    pltpu.sync_copy(batch_hbm, batch_v)

    def _fill_z(i, _):
        zb_v[pl.ds(i * 16, 16)] = jnp.zeros((16,), jnp.float32)
        return _
    lax.fori_loop(0, ZCHUNK // 16, _fill_z, None)
    def _fill_o(i, _):
        ones_v[pl.ds(i * 16, 16)] = jnp.ones((16,), jnp.float32)
        return _
    lax.fori_loop(0, 128 // 16, _fill_o, None)

    nfull = ZONE // ZCHUNK
    def _zero(k, _):
        pltpu.sync_copy(zb_v, hist_sh.at[pl.ds(s * ZONE + k * ZCHUNK, ZCHUNK)])
        return _
    lax.fori_loop(0, nfull, _zero, None)
    tail = ZONE - nfull * ZCHUNK
    if tail:
        pltpu.sync_copy(zb_v.at[pl.ds(0, tail)],
                        hist_sh.at[pl.ds(s * ZONE + nfull * ZCHUNK, tail)])

    pad_idx = jnp.full((16,), C_SIZE, jnp.int32) + wid * 4
    def _fill_pad(i, _):
        idx_v.at[ROWS_PER_HALF - 1][pl.ds(i * 16, 16)] = pad_idx
        idx_v.at[IDX_ROWS - 1][pl.ds(i * 16, 16)] = pad_idx
        return _
    lax.fori_loop(0, 128 // 16, _fill_pad, None)

    plsc.subcore_barrier()

    def _chunk(k, _):
        pltpu.sync_copy(src_hbm.at[pl.ds(base + k * CH, CH)], src_v)
        pltpu.sync_copy(dst_hbm.at[pl.ds(base + k * CH, CH)], dst_v)

        def _index(i, _):
            sv = src_v[pl.ds(i * 16, 16)]
            dv = dst_v[pl.ds(i * 16, 16)]
            gv = plsc.load_gather(batch_v, [sv])
            row = i // 8
            col = (i % 8) * 16
            idx_v.at[row][pl.ds(col, 16)] = gv * N_NODES + sv
            idx_v.at[ROWS_PER_HALF + row][pl.ds(col, 16)] = (gv + N_GRAPHS) * N_NODES + dv
            return _
        lax.fori_loop(0, CVREG, _index, None)

        def _scatter(j, _):
            pltpu.sync_copy(ones_v, hist_sh.at[idx_v.at[j]], add=True)
            return _
        lax.fori_loop(0, IDX_ROWS, _scatter, None)
        return _
    lax.fori_loop(0, NCHUNK, _chunk, None)

    plsc.subcore_barrier()

    pltpu.sync_copy(hist_sh.at[pl.ds(s * ZONE, ZONE)],
                    out_hbm.at[c, pl.ds(s * ZONE, ZONE)])


@jax.jit
def _sc_build_counts(src, dst, batch):
    mesh = plsc.VectorSubcoreMesh(core_axis_name="c", subcore_axis_name="s")
    f = pl.kernel(
        _sc_body,
        out_type=jax.ShapeDtypeStruct((NC, C_SIZE), jnp.float32),
        mesh=mesh,
        compiler_params=pltpu.CompilerParams(needs_layout_passes=False),
        scratch_types=[
            pltpu.VMEM_SHARED((S_SIZE,), jnp.float32),
            pltpu.VMEM((CH,), jnp.int32),
            pltpu.VMEM((CH,), jnp.int32),
            pltpu.VMEM((N_NODES,), jnp.int32),
            pltpu.VMEM((IDX_ROWS, 128), jnp.int32),
            pltpu.VMEM((128,), jnp.float32),
            pltpu.VMEM((ZCHUNK,), jnp.float32),
        ],
    )
    return f(src, dst, batch)


def _tc_body(P_ref, x_ref, W0_ref, b0_ref, W1_ref, b1_ref, W2_ref, b2_ref, o_ref):
    hi = lax.Precision.HIGHEST
    A = P_ref[0] + P_ref[1]
    Y = lax.dot_general(A, x_ref[...], (((1,), (0,)), ((), ())), precision=hi)
    r = jnp.sum(A, axis=1, keepdims=True)
    P1 = lax.dot_general(Y, W0_ref[...], (((1,), (1,)), ((), ())), precision=hi) + r * b0_ref[...]
    P2 = lax.dot_general(P1, W1_ref[...], (((1,), (1,)), ((), ())), precision=hi) + r * b1_ref[...]
    P3 = lax.dot_general(P2, W2_ref[...], (((1,), (1,)), ((), ())), precision=hi) + r * b2_ref[...]
    denom = jnp.maximum(r[:N_GRAPHS, :], 1.0)
    out = jnp.concatenate(
        [P1[:N_GRAPHS], P1[N_GRAPHS:], P2[:N_GRAPHS], P2[N_GRAPHS:],
         P3[:N_GRAPHS], P3[N_GRAPHS:]], axis=1)
    o_ref[...] = out / denom


@jax.jit
def _tc_finish(P, x, W0, b0, W1, b1, W2, b2):
    return pl.pallas_call(
        _tc_body,
        out_shape=jax.ShapeDtypeStruct((N_GRAPHS, 6 * D), jnp.float32),
    )(P, x, W0, b0.reshape(1, D), W1, b1.reshape(1, D), W2, b2.reshape(1, D))


def kernel(x, edge_index, batch, W0, b0, W1, b1, W2, b2):
    src = edge_index[0].astype(jnp.int32)
    dst = edge_index[1].astype(jnp.int32)
    batch32 = batch.astype(jnp.int32)
    P = _sc_build_counts(src, dst, batch32)
    P = P.reshape(NC, 2 * N_GRAPHS, N_NODES)
    return _tc_finish(P, x, W0, b0, W1, b1, W2, b2)


# speedup vs baseline: 82.5714x; 1.1618x over previous
import functools

import jax
import jax.numpy as jnp
from jax import lax
from jax.experimental import pallas as pl
from jax.experimental.pallas import tpu as pltpu
from jax.experimental.pallas import tpu_sc as plsc

N_NODES = 10000
N_EDGES = 320000
N_GRAPHS = 64
D = 128

NC = 2
NS = 16
NW = NC * NS
EPW = N_EDGES // NW
CH = 2000
NCHUNK = EPW // CH
CVREG = CH // 16
HALF = 2048
IDX_N = 2 * HALF
C_SIZE = 2 * N_GRAPHS * N_NODES
S_SIZE = C_SIZE + 128
ZONE = C_SIZE // NS
ZCHUNK = 8000


def _sc_body(src_hbm, dst_hbm, batch_hbm, out_hbm,
             hist_sh, src0_v, src1_v, dst0_v, dst1_v, batch_v,
             idx0_v, idx1_v, ones_v, zb_v, zsem, ssem):
    c = lax.axis_index("c")
    s = lax.axis_index("s")
    wid = s * NC + c
    base = wid * EPW

    def _fill_z(i, _):
        zb_v[pl.ds(i * 16, 16)] = jnp.zeros((16,), jnp.float32)
        return _
    lax.fori_loop(0, ZCHUNK // 16, _fill_z, None)
    def _fill_o(i, _):
        ones_v[pl.ds(i * 16, 16)] = jnp.ones((16,), jnp.float32)
        return _
    lax.fori_loop(0, IDX_N // 16, _fill_o, None)

    nzero = ZONE // ZCHUNK
    def _zero(k, _):
        pltpu.async_copy(zb_v, hist_sh.at[pl.ds(s * ZONE + k * ZCHUNK, ZCHUNK)],
                         zsem)
        return _
    lax.fori_loop(0, nzero, _zero, None)

    pltpu.sync_copy(batch_hbm, batch_v)
    pad_idx = jnp.full((16,), C_SIZE, jnp.int32) + wid * 4
    def _fill_pad(i, _):
        idx0_v[pl.ds(CH + i * 16, 16)] = pad_idx
        idx0_v[pl.ds(HALF + CH + i * 16, 16)] = pad_idx
        idx1_v[pl.ds(CH + i * 16, 16)] = pad_idx
        idx1_v[pl.ds(HALF + CH + i * 16, 16)] = pad_idx
        return _
    lax.fori_loop(0, (HALF - CH) // 16, _fill_pad, None)

    bufs = [(src0_v, dst0_v, idx0_v), (src1_v, dst1_v, idx1_v)]

    def _stage(k):
        sv, dv, _ = bufs[k % 2]
        pltpu.sync_copy(src_hbm.at[pl.ds(base + k * CH, CH)], sv)
        pltpu.sync_copy(dst_hbm.at[pl.ds(base + k * CH, CH)], dv)

    def _compute(k):
        sbuf, dbuf, ibuf = bufs[k % 2]
        def _index(i, _):
            sv = sbuf[pl.ds(i * 16, 16)]
            dv = dbuf[pl.ds(i * 16, 16)]
            gv = plsc.load_gather(batch_v, [sv])
            ibuf[pl.ds(i * 16, 16)] = gv * N_NODES + sv
            ibuf[pl.ds(HALF + i * 16, 16)] = (gv + N_GRAPHS) * N_NODES + dv
            return _
        lax.fori_loop(0, CVREG, _index, None)

    _stage(0)
    _compute(0)

    def _drain_z(k, _):
        pltpu.make_async_copy(
            zb_v, hist_sh.at[pl.ds(s * ZONE + k * ZCHUNK, ZCHUNK)], zsem).wait()
        return _
    lax.fori_loop(0, nzero, _drain_z, None)
    plsc.subcore_barrier()

    def _scatter_start(k):
        ibuf = bufs[k % 2][2]
        pltpu.async_copy(ones_v, hist_sh.at[ibuf], ssem, add=True)

    def _scatter_wait(k):
        ibuf = bufs[k % 2][2]
        pltpu.make_async_copy(ones_v, hist_sh.at[ibuf], ssem).wait()

    for k in range(NCHUNK):
        _scatter_start(k)
        if k + 1 < NCHUNK:
            _stage(k + 1)
            _compute(k + 1)
        _scatter_wait(k)

    plsc.subcore_barrier()

    pltpu.sync_copy(hist_sh.at[pl.ds(s * ZONE, ZONE)],
                    out_hbm.at[c, pl.ds(s * ZONE, ZONE)])


@jax.jit
def _sc_build_counts(src, dst, batch):
    mesh = plsc.VectorSubcoreMesh(core_axis_name="c", subcore_axis_name="s")
    f = pl.kernel(
        _sc_body,
        out_type=jax.ShapeDtypeStruct((NC, C_SIZE), jnp.float32),
        mesh=mesh,
        compiler_params=pltpu.CompilerParams(needs_layout_passes=False),
        scratch_types=[
            pltpu.VMEM_SHARED((S_SIZE,), jnp.float32),
            pltpu.VMEM((CH,), jnp.int32),
            pltpu.VMEM((CH,), jnp.int32),
            pltpu.VMEM((CH,), jnp.int32),
            pltpu.VMEM((CH,), jnp.int32),
            pltpu.VMEM((N_NODES,), jnp.int32),
            pltpu.VMEM((IDX_N,), jnp.int32),
            pltpu.VMEM((IDX_N,), jnp.int32),
            pltpu.VMEM((IDX_N,), jnp.float32),
            pltpu.VMEM((ZCHUNK,), jnp.float32),
            pltpu.SemaphoreType.DMA,
            pltpu.SemaphoreType.DMA,
        ],
    )
    return f(src, dst, batch)


def _tc_body(P_ref, x_ref, W0_ref, b0_ref, W1_ref, b1_ref, W2_ref, b2_ref, o_ref):
    hi = lax.Precision.HIGHEST
    A = P_ref[0] + P_ref[1]
    Y = lax.dot_general(A, x_ref[...], (((1,), (0,)), ((), ())), precision=hi)
    r = jnp.sum(A, axis=1, keepdims=True)
    P1 = lax.dot_general(Y, W0_ref[...], (((1,), (1,)), ((), ())), precision=hi) + r * b0_ref[...]
    P2 = lax.dot_general(P1, W1_ref[...], (((1,), (1,)), ((), ())), precision=hi) + r * b1_ref[...]
    P3 = lax.dot_general(P2, W2_ref[...], (((1,), (1,)), ((), ())), precision=hi) + r * b2_ref[...]
    denom = jnp.maximum(r[:N_GRAPHS, :], 1.0)
    out = jnp.concatenate(
        [P1[:N_GRAPHS], P1[N_GRAPHS:], P2[:N_GRAPHS], P2[N_GRAPHS:],
         P3[:N_GRAPHS], P3[N_GRAPHS:]], axis=1)
    o_ref[...] = out / denom


@jax.jit
def _tc_finish(P, x, W0, b0, W1, b1, W2, b2):
    return pl.pallas_call(
        _tc_body,
        out_shape=jax.ShapeDtypeStruct((N_GRAPHS, 6 * D), jnp.float32),
    )(P, x, W0, b0.reshape(1, D), W1, b1.reshape(1, D), W2, b2.reshape(1, D))


def kernel(x, edge_index, batch, W0, b0, W1, b1, W2, b2):
    src = edge_index[0].astype(jnp.int32)
    dst = edge_index[1].astype(jnp.int32)
    batch32 = batch.astype(jnp.int32)
    P = _sc_build_counts(src, dst, batch32)
    P = P.reshape(NC, 2 * N_GRAPHS, N_NODES)
    return _tc_finish(P, x, W0, b0, W1, b1, W2, b2)


# speedup vs baseline: 141.2557x; 1.7107x over previous
import functools

import jax
import jax.numpy as jnp
from jax import lax
from jax.experimental import pallas as pl
from jax.experimental.pallas import tpu as pltpu
from jax.experimental.pallas import tpu_sc as plsc

N_NODES = 10000
N_EDGES = 320000
N_GRAPHS = 64
D = 128

NC = 2
NS = 16
NW = NC * NS
EPW = N_EDGES // NW
CH = 2000
NCHUNK = EPW // CH
CVREG = CH // 16
HALF = 2048
IDX_N = 2 * HALF
STRIDE = 10240
C_SIZE = 2 * N_GRAPHS * STRIDE
S_SIZE = C_SIZE + 128
ZONE = C_SIZE // NS
ZCHUNK = 8192


def _sc_body(ei_hbm, batch_hbm, out_hbm,
             hist_sh, src0_v, src1_v, dst0_v, dst1_v, batch_v,
             idx0_v, idx1_v, ones_v, zb_v, zsem, ssem):
    c = lax.axis_index("c")
    s = lax.axis_index("s")
    wid = s * NC + c
    base = wid * EPW

    def _fill_z(i, _):
        zb_v[pl.ds(i * 16, 16)] = jnp.zeros((16,), jnp.float32)
        return _
    lax.fori_loop(0, ZCHUNK // 16, _fill_z, None)
    def _fill_o(i, _):
        ones_v[pl.ds(i * 16, 16)] = jnp.ones((16,), jnp.float32)
        return _
    lax.fori_loop(0, IDX_N // 16, _fill_o, None)

    nzero = ZONE // ZCHUNK
    def _zero(k, _):
        pltpu.async_copy(zb_v, hist_sh.at[pl.ds(s * ZONE + k * ZCHUNK, ZCHUNK)],
                         zsem)
        return _
    lax.fori_loop(0, nzero, _zero, None)

    pltpu.sync_copy(batch_hbm, batch_v)
    pad_idx = jnp.full((16,), C_SIZE, jnp.int32) + wid * 4
    def _fill_pad(i, _):
        idx0_v[pl.ds(CH + i * 16, 16)] = pad_idx
        idx0_v[pl.ds(HALF + CH + i * 16, 16)] = pad_idx
        idx1_v[pl.ds(CH + i * 16, 16)] = pad_idx
        idx1_v[pl.ds(HALF + CH + i * 16, 16)] = pad_idx
        return _
    lax.fori_loop(0, (HALF - CH) // 16, _fill_pad, None)

    bufs = [(src0_v, dst0_v, idx0_v), (src1_v, dst1_v, idx1_v)]

    def _stage(k):
        sv, dv, _ = bufs[k % 2]
        pltpu.sync_copy(ei_hbm.at[pl.ds(base + k * CH, CH)], sv)
        pltpu.sync_copy(ei_hbm.at[pl.ds(N_EDGES + base + k * CH, CH)], dv)

    def _compute(k):
        sbuf, dbuf, ibuf = bufs[k % 2]
        def _index(i, _):
            sv = sbuf[pl.ds(i * 16, 16)]
            dv = dbuf[pl.ds(i * 16, 16)]
            gv = plsc.load_gather(batch_v, [sv])
            ibuf[pl.ds(i * 16, 16)] = gv * STRIDE + sv
            ibuf[pl.ds(HALF + i * 16, 16)] = (gv + N_GRAPHS) * STRIDE + dv
            return _
        lax.fori_loop(0, CVREG, _index, None)

    _stage(0)
    _compute(0)

    def _drain_z(k, _):
        pltpu.make_async_copy(
            zb_v, hist_sh.at[pl.ds(s * ZONE + k * ZCHUNK, ZCHUNK)], zsem).wait()
        return _
    lax.fori_loop(0, nzero, _drain_z, None)
    plsc.subcore_barrier()

    def _scatter_start(k):
        ibuf = bufs[k % 2][2]
        pltpu.async_copy(ones_v, hist_sh.at[ibuf], ssem, add=True)

    def _scatter_wait(k):
        ibuf = bufs[k % 2][2]
        pltpu.make_async_copy(ones_v, hist_sh.at[ibuf], ssem).wait()

    for k in range(NCHUNK):
        _scatter_start(k)
        if k + 1 < NCHUNK:
            _stage(k + 1)
            _compute(k + 1)
        _scatter_wait(k)

    plsc.subcore_barrier()

    for j in range(2 * N_GRAPHS // NS):
        row = s * (2 * N_GRAPHS // NS) + j
        pltpu.sync_copy(hist_sh.at[pl.ds(row * STRIDE, STRIDE)],
                        out_hbm.at[c, row])


@jax.jit
def _sc_build_counts(ei, batch):
    mesh = plsc.VectorSubcoreMesh(core_axis_name="c", subcore_axis_name="s")
    f = pl.kernel(
        _sc_body,
        out_type=jax.ShapeDtypeStruct((NC, 2 * N_GRAPHS, STRIDE), jnp.float32),
        mesh=mesh,
        compiler_params=pltpu.CompilerParams(needs_layout_passes=False),
        scratch_types=[
            pltpu.VMEM_SHARED((S_SIZE,), jnp.float32),
            pltpu.VMEM((CH,), jnp.int32),
            pltpu.VMEM((CH,), jnp.int32),
            pltpu.VMEM((CH,), jnp.int32),
            pltpu.VMEM((CH,), jnp.int32),
            pltpu.VMEM((N_NODES,), jnp.int32),
            pltpu.VMEM((IDX_N,), jnp.int32),
            pltpu.VMEM((IDX_N,), jnp.int32),
            pltpu.VMEM((IDX_N,), jnp.float32),
            pltpu.VMEM((ZCHUNK,), jnp.float32),
            pltpu.SemaphoreType.DMA,
            pltpu.SemaphoreType.DMA,
        ],
    )
    return f(ei, batch)


def _tc_body(P_ref, x_ref, W0_ref, b0_ref, W1_ref, b1_ref, W2_ref, b2_ref, o_ref):
    hi = lax.Precision.HIGHEST
    A = P_ref[0] + P_ref[1]
    Y = lax.dot_general(A[:, :N_NODES], x_ref[...],
                        (((1,), (0,)), ((), ())), precision=hi)
    r = jnp.sum(A, axis=1, keepdims=True)
    P1 = lax.dot_general(Y, W0_ref[...], (((1,), (1,)), ((), ())), precision=hi) + r * b0_ref[...]
    P2 = lax.dot_general(P1, W1_ref[...], (((1,), (1,)), ((), ())), precision=hi) + r * b1_ref[...]
    P3 = lax.dot_general(P2, W2_ref[...], (((1,), (1,)), ((), ())), precision=hi) + r * b2_ref[...]
    denom = jnp.maximum(r[:N_GRAPHS, :], 1.0)
    out = jnp.concatenate(
        [P1[:N_GRAPHS], P1[N_GRAPHS:], P2[:N_GRAPHS], P2[N_GRAPHS:],
         P3[:N_GRAPHS], P3[N_GRAPHS:]], axis=1)
    o_ref[...] = out / denom


@jax.jit
def _tc_finish(P, x, W0, b0, W1, b1, W2, b2):
    return pl.pallas_call(
        _tc_body,
        out_shape=jax.ShapeDtypeStruct((N_GRAPHS, 6 * D), jnp.float32),
    )(P, x, W0, b0.reshape(1, D), W1, b1.reshape(1, D), W2, b2.reshape(1, D))


def kernel(x, edge_index, batch, W0, b0, W1, b1, W2, b2):
    ei_flat = edge_index.astype(jnp.int32).reshape(2 * N_EDGES)
    P = _sc_build_counts(ei_flat, batch.astype(jnp.int32))
    return _tc_finish(P, x, W0, b0, W1, b1, W2, b2)


# speedup vs baseline: 165.2195x; 1.1696x over previous
import functools

import jax
import jax.numpy as jnp
from jax import lax
from jax.experimental import pallas as pl
from jax.experimental.pallas import tpu as pltpu
from jax.experimental.pallas import tpu_sc as plsc

N_NODES = 10000
N_EDGES = 320000
N_GRAPHS = 64
D = 128

NC = 2
NS = 16
NW = NC * NS
EPW = N_EDGES // NW
CH = 2000
NCHUNK = EPW // CH
CVREG = CH // 16
HALF = 2048
IDX_N = 2 * HALF
STRIDE = 10240
C_SIZE = (N_GRAPHS + 1) * STRIDE
S_SIZE = C_SIZE + 128
ZONE = C_SIZE // NS
ZCHUNK = 8320


def _sc_body(ei_hbm, batch_hbm, out_hbm, deg_hbm,
             hist_sh, src0_v, src1_v, dst0_v, dst1_v, batch_v,
             idx0_v, idx1_v, ones_v, zb_v, zsem, ssem):
    c = lax.axis_index("c")
    s = lax.axis_index("s")
    wid = s * NC + c
    base = wid * EPW

    def _fill_z(i, _):
        zb_v[pl.ds(i * 16, 16)] = jnp.zeros((16,), jnp.float32)
        return _
    lax.fori_loop(0, ZCHUNK // 16, _fill_z, None)
    def _fill_o(i, _):
        ones_v[pl.ds(i * 16, 16)] = jnp.ones((16,), jnp.float32)
        return _
    lax.fori_loop(0, IDX_N // 16, _fill_o, None)

    nzero = ZONE // ZCHUNK
    def _zero(k, _):
        pltpu.async_copy(zb_v, hist_sh.at[pl.ds(s * ZONE + k * ZCHUNK, ZCHUNK)],
                         zsem)
        return _
    lax.fori_loop(0, nzero, _zero, None)

    pltpu.sync_copy(batch_hbm, batch_v)
    pad_idx = jnp.full((16,), C_SIZE, jnp.int32) + wid * 4
    def _fill_pad(i, _):
        idx0_v[pl.ds(CH + i * 16, 16)] = pad_idx
        idx0_v[pl.ds(HALF + CH + i * 16, 16)] = pad_idx
        idx1_v[pl.ds(CH + i * 16, 16)] = pad_idx
        idx1_v[pl.ds(HALF + CH + i * 16, 16)] = pad_idx
        return _
    lax.fori_loop(0, (HALF - CH) // 16, _fill_pad, None)

    bufs = [(src0_v, dst0_v, idx0_v), (src1_v, dst1_v, idx1_v)]

    def _stage(k):
        sv, dv, _ = bufs[k % 2]
        pltpu.sync_copy(ei_hbm.at[pl.ds(base + k * CH, CH)], sv)
        pltpu.sync_copy(ei_hbm.at[pl.ds(N_EDGES + base + k * CH, CH)], dv)

    def _compute(k):
        sbuf, dbuf, ibuf = bufs[k % 2]
        def _index(i, _):
            sv = sbuf[pl.ds(i * 16, 16)]
            dv = dbuf[pl.ds(i * 16, 16)]
            gv = plsc.load_gather(batch_v, [sv])
            ibuf[pl.ds(i * 16, 16)] = sv
            ibuf[pl.ds(HALF + i * 16, 16)] = (gv + 1) * STRIDE + dv
            return _
        lax.fori_loop(0, CVREG, _index, None)

    _stage(0)
    _compute(0)

    def _drain_z(k, _):
        pltpu.make_async_copy(
            zb_v, hist_sh.at[pl.ds(s * ZONE + k * ZCHUNK, ZCHUNK)], zsem).wait()
        return _
    lax.fori_loop(0, nzero, _drain_z, None)
    plsc.subcore_barrier()

    def _scatter_start(k):
        ibuf = bufs[k % 2][2]
        pltpu.async_copy(ones_v, hist_sh.at[ibuf], ssem, add=True)

    def _scatter_wait(k):
        ibuf = bufs[k % 2][2]
        pltpu.make_async_copy(ones_v, hist_sh.at[ibuf], ssem).wait()

    for k in range(NCHUNK):
        _scatter_start(k)
        if k + 1 < NCHUNK:
            _stage(k + 1)
            _compute(k + 1)
        _scatter_wait(k)

    plsc.subcore_barrier()

    for j in range(N_GRAPHS // NS):
        row = s * (N_GRAPHS // NS) + j
        pltpu.sync_copy(hist_sh.at[pl.ds((1 + row) * STRIDE, STRIDE)],
                        out_hbm.at[c, row])
    @pl.when(s == 0)
    def _():
        pltpu.sync_copy(hist_sh.at[pl.ds(0, STRIDE)], deg_hbm.at[c, 0])


@jax.jit
def _sc_build_counts(ei, batch):
    mesh = plsc.VectorSubcoreMesh(core_axis_name="c", subcore_axis_name="s")
    f = pl.kernel(
        _sc_body,
        out_type=(jax.ShapeDtypeStruct((NC, N_GRAPHS, STRIDE), jnp.float32),
                  jax.ShapeDtypeStruct((NC, 1, STRIDE), jnp.float32)),
        mesh=mesh,
        compiler_params=pltpu.CompilerParams(needs_layout_passes=False),
        scratch_types=[
            pltpu.VMEM_SHARED((S_SIZE,), jnp.float32),
            pltpu.VMEM((CH,), jnp.int32),
            pltpu.VMEM((CH,), jnp.int32),
            pltpu.VMEM((CH,), jnp.int32),
            pltpu.VMEM((CH,), jnp.int32),
            pltpu.VMEM((N_NODES,), jnp.int32),
            pltpu.VMEM((IDX_N,), jnp.int32),
            pltpu.VMEM((IDX_N,), jnp.int32),
            pltpu.VMEM((IDX_N,), jnp.float32),
            pltpu.VMEM((ZCHUNK,), jnp.float32),
            pltpu.SemaphoreType.DMA,
            pltpu.SemaphoreType.DMA,
        ],
    )
    return f(ei, batch)


def _tc_body(P_ref, D_ref, b_ref, x_ref,
             W0_ref, b0_ref, W1_ref, b1_ref, W2_ref, b2_ref, o_ref):
    hi = lax.Precision.DEFAULT
    deg = D_ref[0, 0, :N_NODES] + D_ref[1, 0, :N_NODES]
    gids = lax.broadcasted_iota(jnp.int32, (N_GRAPHS, N_NODES), 0)
    Csrc = jnp.where(b_ref[...] == gids, deg[None, :], 0.0)
    Adst = P_ref[0] + P_ref[1]
    Ysrc = lax.dot_general(Csrc, x_ref[...], (((1,), (0,)), ((), ())),
                           precision=hi)
    Ydst = lax.dot_general(Adst[:, :N_NODES], x_ref[...],
                           (((1,), (0,)), ((), ())), precision=hi)
    Y = jnp.concatenate([Ysrc, Ydst], axis=0)
    rs = jnp.sum(Csrc, axis=1, keepdims=True)
    r = jnp.concatenate([rs, jnp.sum(Adst, axis=1, keepdims=True)], axis=0)
    P1 = lax.dot_general(Y, W0_ref[...], (((1,), (1,)), ((), ())), precision=hi) + r * b0_ref[...]
    P2 = lax.dot_general(P1, W1_ref[...], (((1,), (1,)), ((), ())), precision=hi) + r * b1_ref[...]
    P3 = lax.dot_general(P2, W2_ref[...], (((1,), (1,)), ((), ())), precision=hi) + r * b2_ref[...]
    denom = jnp.maximum(rs, 1.0)
    out = jnp.concatenate(
        [P1[:N_GRAPHS], P1[N_GRAPHS:], P2[:N_GRAPHS], P2[N_GRAPHS:],
         P3[:N_GRAPHS], P3[N_GRAPHS:]], axis=1)
    o_ref[...] = out / denom


@jax.jit
def _tc_finish(P, Dg, batch2d, x, W0, b0, W1, b1, W2, b2):
    return pl.pallas_call(
        _tc_body,
        out_shape=jax.ShapeDtypeStruct((N_GRAPHS, 6 * D), jnp.float32),
    )(P, Dg, batch2d, x, W0, b0.reshape(1, D), W1, b1.reshape(1, D),
      W2, b2.reshape(1, D))


def kernel(x, edge_index, batch, W0, b0, W1, b1, W2, b2):
    ei_flat = edge_index.astype(jnp.int32).reshape(2 * N_EDGES)
    batch32 = batch.astype(jnp.int32)
    P, Dg = _sc_build_counts(ei_flat, batch32)
    return _tc_finish(P, Dg, batch32.reshape(1, N_NODES), x, W0, b0, W1, b1, W2, b2)


# speedup vs baseline: 180.7113x; 1.0938x over previous
import functools

import jax
import jax.numpy as jnp
from jax import lax
from jax.experimental import pallas as pl
from jax.experimental.pallas import tpu as pltpu
from jax.experimental.pallas import tpu_sc as plsc

N_NODES = 10000
N_EDGES = 320000
N_GRAPHS = 64
D = 128

NC = 2
NS = 16
NW = NC * NS
EPW = N_EDGES // NW
CH = 2048
NCHUNK = 5
HALF = 2048
IDX_N = 2 * HALF
STRIDE = 10240
C_SIZE = (N_GRAPHS + 1) * STRIDE
S_SIZE = C_SIZE + 128
ZONE = C_SIZE // NS
ZCHUNK = 8320


def _sc_body(ei_hbm, batch_hbm, out_hbm, deg_hbm,
             hist_sh, sd0_v, sd1_v, batch_v,
             idx0_v, idx1_v, ones_v, zb_v, zsem, ssem):
    c = lax.axis_index("c")
    s = lax.axis_index("s")
    wid = s * NC + c
    off0 = wid * EPW
    wstart = jnp.minimum(off0 - off0 % 128, N_EDGES - NCHUNK * CH)
    skew = off0 - wstart

    def _fill_z(i, _):
        zb_v[pl.ds(i * 16, 16)] = jnp.zeros((16,), jnp.float32)
        return _
    lax.fori_loop(0, ZCHUNK // 16, _fill_z, None)
    def _fill_o(i, _):
        ones_v[pl.ds(i * 16, 16)] = jnp.ones((16,), jnp.float32)
        return _
    lax.fori_loop(0, IDX_N // 16, _fill_o, None)

    nzero = ZONE // ZCHUNK
    def _zero(k, _):
        pltpu.async_copy(zb_v, hist_sh.at[pl.ds(s * ZONE + k * ZCHUNK, ZCHUNK)],
                         zsem)
        return _
    lax.fori_loop(0, nzero, _zero, None)

    pltpu.sync_copy(batch_hbm, batch_v)
    pad_idx = jnp.full((16,), C_SIZE, jnp.int32) + wid * 4

    bufs = [(sd0_v, idx0_v), (sd1_v, idx1_v)]

    def _stage(k, b):
        sd = bufs[b][0]
        pltpu.sync_copy(
            ei_hbm.at[:, pl.ds(pl.multiple_of(wstart + k * CH, 128), CH)], sd)

    def _compute(k, b):
        sd = bufs[b][0]
        ibuf = bufs[b][1]
        lo = jnp.where(k == 0, skew // 16, 0)
        hi = jnp.minimum(HALF // 16, (skew + EPW - k * CH) // 16)
        zrow = jnp.zeros((16,), jnp.int32)
        orow = jnp.ones((16,), jnp.int32)
        lane = lax.iota(jnp.int32, 16)
        def _index(i, _):
            col = i * 16 + lane
            sv = plsc.load_gather(sd, [zrow, col])
            dv = plsc.load_gather(sd, [orow, col])
            gv = plsc.load_gather(batch_v, [sv])
            ibuf[pl.ds(i * 16, 16)] = sv
            ibuf[pl.ds(HALF + i * 16, 16)] = (gv + 1) * STRIDE + dv
            return _
        lax.fori_loop(lo, hi, _index, None)
        def _fill_pad(i, _):
            ibuf[pl.ds(i * 16, 16)] = pad_idx
            ibuf[pl.ds(HALF + i * 16, 16)] = pad_idx
            return _
        lax.fori_loop(0, lo, _fill_pad, None)
        lax.fori_loop(hi, HALF // 16, _fill_pad, None)

    _stage(0, 0)
    _compute(0, 0)

    def _drain_z(k, _):
        pltpu.make_async_copy(
            zb_v, hist_sh.at[pl.ds(s * ZONE + k * ZCHUNK, ZCHUNK)], zsem).wait()
        return _
    lax.fori_loop(0, nzero, _drain_z, None)
    plsc.subcore_barrier()

    def _scatter_start(b):
        ibuf = bufs[b][1]
        pltpu.async_copy(ones_v, hist_sh.at[ibuf], ssem, add=True)

    def _scatter_wait(b):
        ibuf = bufs[b][1]
        pltpu.make_async_copy(ones_v, hist_sh.at[ibuf], ssem).wait()

    def _step(k, b):
        _scatter_start(b)
        _stage(k + 1, 1 - b)
        _compute(k + 1, 1 - b)
        _scatter_wait(b)

    def _loop(k, _):
        @pl.when(k % 2 == 0)
        def _():
            _step(k, 0)
        @pl.when(k % 2 == 1)
        def _():
            _step(k, 1)
        return _
    lax.fori_loop(0, NCHUNK - 1, _loop, None)
    last = (NCHUNK - 1) % 2
    _scatter_start(last)
    _scatter_wait(last)

    plsc.subcore_barrier()

    for j in range(N_GRAPHS // NS):
        row = s * (N_GRAPHS // NS) + j
        pltpu.sync_copy(hist_sh.at[pl.ds((1 + row) * STRIDE, STRIDE)],
                        out_hbm.at[c, row])
    @pl.when(s == 0)
    def _():
        pltpu.sync_copy(hist_sh.at[pl.ds(0, STRIDE)], deg_hbm.at[c, 0])


@jax.jit
def _sc_build_counts(ei, batch):
    mesh = plsc.VectorSubcoreMesh(core_axis_name="c", subcore_axis_name="s")
    f = pl.kernel(
        _sc_body,
        out_type=(jax.ShapeDtypeStruct((NC, N_GRAPHS, STRIDE), jnp.float32),
                  jax.ShapeDtypeStruct((NC, 1, STRIDE), jnp.float32)),
        mesh=mesh,
        compiler_params=pltpu.CompilerParams(needs_layout_passes=False),
        scratch_types=[
            pltpu.VMEM_SHARED((S_SIZE,), jnp.float32),
            pltpu.VMEM((2, CH), jnp.int32),
            pltpu.VMEM((2, CH), jnp.int32),
            pltpu.VMEM((N_NODES,), jnp.int32),
            pltpu.VMEM((IDX_N,), jnp.int32),
            pltpu.VMEM((IDX_N,), jnp.int32),
            pltpu.VMEM((IDX_N,), jnp.float32),
            pltpu.VMEM((ZCHUNK,), jnp.float32),
            pltpu.SemaphoreType.DMA,
            pltpu.SemaphoreType.DMA,
        ],
    )
    return f(ei, batch)


def _tc_body(P_ref, D_ref, b_ref, x_ref,
             W0_ref, b0_ref, W1_ref, b1_ref, W2_ref, b2_ref, o_ref):
    hi = lax.Precision.DEFAULT
    deg = D_ref[0, 0, :N_NODES] + D_ref[1, 0, :N_NODES]
    gids = lax.broadcasted_iota(jnp.int32, (N_GRAPHS, N_NODES), 0)
    Csrc = jnp.where(b_ref[...] == gids, deg[None, :], 0.0)
    Adst = P_ref[0] + P_ref[1]
    Ysrc = lax.dot_general(Csrc, x_ref[...], (((1,), (0,)), ((), ())),
                           precision=hi)
    Ydst = lax.dot_general(Adst[:, :N_NODES], x_ref[...],
                           (((1,), (0,)), ((), ())), precision=hi)
    Y = jnp.concatenate([Ysrc, Ydst], axis=0)
    rs = jnp.sum(Csrc, axis=1, keepdims=True)
    r = jnp.concatenate([rs, jnp.sum(Adst, axis=1, keepdims=True)], axis=0)
    P1 = lax.dot_general(Y, W0_ref[...], (((1,), (1,)), ((), ())), precision=hi) + r * b0_ref[...]
    P2 = lax.dot_general(P1, W1_ref[...], (((1,), (1,)), ((), ())), precision=hi) + r * b1_ref[...]
    P3 = lax.dot_general(P2, W2_ref[...], (((1,), (1,)), ((), ())), precision=hi) + r * b2_ref[...]
    denom = jnp.maximum(rs, 1.0)
    out = jnp.concatenate(
        [P1[:N_GRAPHS], P1[N_GRAPHS:], P2[:N_GRAPHS], P2[N_GRAPHS:],
         P3[:N_GRAPHS], P3[N_GRAPHS:]], axis=1)
    o_ref[...] = out / denom


@jax.jit
def _tc_finish(P, Dg, batch2d, x, W0, b0, W1, b1, W2, b2):
    return pl.pallas_call(
        _tc_body,
        out_shape=jax.ShapeDtypeStruct((N_GRAPHS, 6 * D), jnp.float32),
    )(P, Dg, batch2d, x, W0, b0.reshape(1, D), W1, b1.reshape(1, D),
      W2, b2.reshape(1, D))


def kernel(x, edge_index, batch, W0, b0, W1, b1, W2, b2):
    batch32 = batch.astype(jnp.int32)
    P, Dg = _sc_build_counts(edge_index.astype(jnp.int32), batch32)
    return _tc_finish(P, Dg, batch32.reshape(1, N_NODES), x, W0, b0, W1, b1, W2, b2)
